# baseline (speedup 1.0000x reference)
import functools

import jax
import jax.numpy as jnp
from jax import lax
from jax.experimental import pallas as pl
from jax.experimental.pallas import tpu as pltpu
from jax.experimental.pallas import tpu_sc as plsc

G = 10000
B = 64
NNZ = 640000
ALPHA = 0.1

NT = 16
EPT = NNZ // NT
CH = 100
NBUF = 4
GAHEAD = 2
IBC = 40
NBLK = EPT // (IBC * CH)
ROWS0 = 632
ROWS_LAST = G - (NT - 1) * ROWS0
ZR = 320

_LANE_DNUMS = lax.GatherDimensionNumbers(
    offset_dims=(), collapsed_slice_dims=(0,), start_index_map=(0,))


def _lane_bcast(v16, el):
    idx = jnp.full((16, 1), el, jnp.int32)
    return lax.gather(v16, idx, _LANE_DNUMS, (1,),
                      mode=lax.GatherScatterMode.PROMISE_IN_BOUNDS)

H1 = 1024
HB = 128
NHB = H1 // HB


def _embed_body(xs_ref, xt_ref, pm_ref, pe_ref, os_ref, ot_ref):
    em = pm_ref[0]
    bm = pm_ref[1]
    ee = pe_ref[0]
    be = pe_ref[1]
    os_ref[...] = jnp.maximum(em * xs_ref[...] + bm, 0.0).T
    ot_ref[...] = jnp.maximum(ee * xt_ref[...] + be, 0.0).T


def _embed(xs2, xt2, pm, pe):
    return pl.pallas_call(
        _embed_body,
        in_specs=[
            pl.BlockSpec((B, G), lambda: (0, 0)),
            pl.BlockSpec((B, G), lambda: (0, 0)),
            pl.BlockSpec(memory_space=pltpu.SMEM),
            pl.BlockSpec(memory_space=pltpu.SMEM),
        ],
        out_specs=[
            pl.BlockSpec((G, B), lambda: (0, 0)),
            pl.BlockSpec((G, B), lambda: (0, 0)),
        ],
        out_shape=[
            jax.ShapeDtypeStruct((G, B), jnp.float32),
            jax.ShapeDtypeStruct((G, B), jnp.float32),
        ],
    )(xs2, xt2, pm, pe)


def _spmm_body(xs, xt, rs, cs, vs, rt, ct, vt, zz, ss, st,
               acc, xsp, cv, rv, vv, buf, sem_g, sem_sc, sem_i):
    cid = lax.axis_index("c")
    sid = lax.axis_index("s")

    def run(rows_hbm, cols_hbm, vals_hbm, x_hbm, out_hbm):
        rstart = pl.multiple_of(sid * ROWS0, 8)

        @pl.when(sid < NT - 1)
        def _():
            pltpu.sync_copy(zz.at[pl.ds(rstart, ROWS0)],
                            acc.at[pl.ds(rstart, ROWS0)])
            pltpu.sync_copy(x_hbm.at[pl.ds(rstart, ROWS0)],
                            xsp.at[pl.ds(rstart, ROWS0)])

        @pl.when(sid == NT - 1)
        def _():
            base = (NT - 1) * ROWS0
            pltpu.sync_copy(zz.at[pl.ds(base, ROWS_LAST)],
                            acc.at[pl.ds(base, ROWS_LAST)])
            pltpu.sync_copy(x_hbm.at[pl.ds(base, ROWS_LAST)],
                            xsp.at[pl.ds(base, ROWS_LAST)])

        plsc.subcore_barrier()

        def scale_chunk(j, b):
            jv = jnp.full((16,), j, jnp.int32)

            @plsc.parallel_loop(0, CH, 1, unroll=5)
            def _body(e):
                vsp = plsc.load_gather(vv, [jv, jnp.full((16,), e, jnp.int32)])
                for k in range(4):
                    sl = pl.ds(k * 16, 16)
                    buf[b, e, sl] = buf[b, e, sl] * vsp

        def wait_gather(b):
            pltpu.make_async_copy(xsp.at[pl.ds(0, CH)], buf.at[b],
                                  sem_g).wait()

        def wait_scatter(b):
            pltpu.make_async_copy(buf.at[b], acc.at[rv.at[0]],
                                  sem_sc).wait()

        def block(blk, carry):
            ci = pltpu.async_copy(cols_hbm.at[sid, blk], cv, sem_i)
            ri = pltpu.async_copy(rows_hbm.at[sid, blk], rv, sem_i)
            vi = pltpu.async_copy(vals_hbm.at[sid, blk], vv, sem_i)
            ci.wait()
            ri.wait()
            vi.wait()
            for p in range(GAHEAD):
                pltpu.async_copy(xsp.at[cv.at[p]], buf.at[p], sem_g)

            def it(i, c2):
                for b in range(NBUF):
                    j = NBUF * i + b
                    wait_gather(b)
                    scale_chunk(j, b)
                    pltpu.async_copy(buf.at[b], acc.at[rv.at[j]], sem_sc,
                                     add=True)
                    b2 = (b + GAHEAD) % NBUF
                    if b < NBUF - GAHEAD:
                        @pl.when(i > 0)
                        def _():
                            wait_scatter(b2)
                        pltpu.async_copy(xsp.at[cv.at[j + GAHEAD]],
                                         buf.at[b2], sem_g)
                    else:
                        wait_scatter(b2)

                        @pl.when(i < IBC // NBUF - 1)
                        def _():
                            pltpu.async_copy(xsp.at[cv.at[j + GAHEAD]],
                                             buf.at[b2], sem_g)
                return c2

            lax.fori_loop(0, IBC // NBUF, it, 0)
            wait_scatter(NBUF - 2)
            wait_scatter(NBUF - 1)
            return carry

        lax.fori_loop(0, NBLK, block, 0)
        plsc.subcore_barrier()

        @pl.when(sid < NT - 1)
        def _():
            pltpu.sync_copy(acc.at[pl.ds(rstart, ROWS0)],
                            out_hbm.at[pl.ds(rstart, ROWS0)])

        @pl.when(sid == NT - 1)
        def _():
            base = (NT - 1) * ROWS0
            pltpu.sync_copy(acc.at[pl.ds(base, ROWS_LAST)],
                            out_hbm.at[pl.ds(base, ROWS_LAST)])

    @pl.when(cid == 0)
    def _():
        run(rs, cs, vs, xs, ss)

    @pl.when(cid == 1)
    def _():
        run(rt, ct, vt, xt, st)


def _spmm(xs0, xt0, rows_s, cols_s, vals_s, rows_t, cols_t, vals_t):
    mesh = plsc.VectorSubcoreMesh(core_axis_name="c", subcore_axis_name="s",
                                  num_cores=2, num_subcores=NT)
    f = pl.kernel(
        _spmm_body,
        out_type=[
            jax.ShapeDtypeStruct((G, B), jnp.float32),
            jax.ShapeDtypeStruct((G, B), jnp.float32),
        ],
        mesh=mesh,
        scratch_types=[
            pltpu.VMEM_SHARED((G, B), jnp.float32),
            pltpu.VMEM_SHARED((G, B), jnp.float32),
            pltpu.VMEM((IBC, CH), jnp.int32),
            pltpu.VMEM((IBC, CH), jnp.int32),
            pltpu.VMEM((IBC, CH), jnp.float32),
            pltpu.VMEM((NBUF, CH, B), jnp.float32),
            pltpu.SemaphoreType.DMA,
            pltpu.SemaphoreType.DMA,
            pltpu.SemaphoreType.DMA,
        ],
        compiler_params=pltpu.CompilerParams(needs_layout_passes=False,
                                             use_tc_tiling_on_sc=False),
    )
    zz = jnp.zeros((G, B), jnp.float32)
    return f(xs0, xt0, rows_s, cols_s, vals_s, rows_t, cols_t, vals_t, zz)


def _mlp_body(ss_ref, st_ref, xs0_ref, xt0_ref, w1_ref, b1_ref,
              w2_ref, b2_ref, w3_ref, b3_ref, out_ref, had_scr, h1_scr):
    k = pl.program_id(0)

    @pl.when(k == 0)
    def _():
        beta = 1.0 - ALPHA
        had_scr[...] = ((beta * ss_ref[...] + ALPHA * xs0_ref[...]) *
                        (beta * st_ref[...] + ALPHA * xt0_ref[...]))

    h1_scr[pl.ds(k * HB, HB), :] = jnp.dot(
        w1_ref[...], had_scr[...], preferred_element_type=jnp.float32)

    @pl.when(k == NHB - 1)
    def _():
        h1 = jnp.maximum(h1_scr[...] + b1_ref[...], 0.0)
        h2 = jnp.maximum(
            jnp.dot(w2_ref[...], h1, preferred_element_type=jnp.float32)
            + b2_ref[...], 0.0)
        out_ref[...] = (
            jnp.dot(w3_ref[...], h2, preferred_element_type=jnp.float32)
            + b3_ref[...])


def _mlp(ss, st, xs0, xt0, W1, b1c, W2, b2c, W3, b3c):
    nc = W3.shape[0]
    return pl.pallas_call(
        _mlp_body,
        grid=(NHB,),
        in_specs=[
            pl.BlockSpec((G, B), lambda i: (0, 0)),
            pl.BlockSpec((G, B), lambda i: (0, 0)),
            pl.BlockSpec((G, B), lambda i: (0, 0)),
            pl.BlockSpec((G, B), lambda i: (0, 0)),
            pl.BlockSpec((HB, G), lambda i: (i, 0)),
            pl.BlockSpec((H1, 1), lambda i: (0, 0)),
            pl.BlockSpec((128, H1), lambda i: (0, 0)),
            pl.BlockSpec((128, 1), lambda i: (0, 0)),
            pl.BlockSpec((nc, 128), lambda i: (0, 0)),
            pl.BlockSpec((nc, 1), lambda i: (0, 0)),
        ],
        out_specs=pl.BlockSpec((nc, B), lambda i: (0, 0)),
        out_shape=jax.ShapeDtypeStruct((nc, B), jnp.float32),
        scratch_shapes=[pltpu.VMEM((G, B), jnp.float32),
                        pltpu.VMEM((H1, B), jnp.float32)],
    )(ss, st, xs0, xt0, W1, b1c, W2, b2c, W3, b3c)


def kernel(x_sample, x_TF, adj_idx, adj_val, adj_t_idx, adj_t_val,
           emb_mut, bias_mut, emb_exp, bias_exp,
           W1, b1, W2, b2, W3, b3):
    xs2 = x_sample.reshape(B, G)
    xt2 = x_TF.reshape(B, G)
    pm = jnp.concatenate([emb_mut, bias_mut]).astype(jnp.float32)
    pe = jnp.concatenate([emb_exp, bias_exp]).astype(jnp.float32)
    xs0, xt0 = _embed(xs2, xt2, pm, pe)

    esh = (NT, NBLK, IBC, CH)
    rows_s = adj_idx[0].astype(jnp.int32).reshape(esh)
    cols_s = adj_idx[1].astype(jnp.int32).reshape(esh)
    rows_t = adj_t_idx[0].astype(jnp.int32).reshape(esh)
    cols_t = adj_t_idx[1].astype(jnp.int32).reshape(esh)
    ss, st = _spmm(xs0, xt0, rows_s, cols_s, adj_val.reshape(esh),
                   rows_t, cols_t, adj_t_val.reshape(esh))

    out = _mlp(ss, st, xs0, xt0,
               W1, b1.reshape(-1, 1), W2, b2.reshape(-1, 1),
               W3, b3.reshape(-1, 1))
    return out.T


# speedup vs baseline: 1.2034x; 1.2034x over previous
import functools

import jax
import jax.numpy as jnp
from jax import lax
from jax.experimental import pallas as pl
from jax.experimental.pallas import tpu as pltpu
from jax.experimental.pallas import tpu_sc as plsc

G = 10000
B = 64
NNZ = 640000
ALPHA = 0.1

NT = 16
EPT = NNZ // NT
CH = 80
NBUF = 4
GAHEAD = 2
IBC = 100
NBLK = EPT // (IBC * CH)
ROWS0 = 632
ROWS_LAST = G - (NT - 1) * ROWS0
ZR = 320

_LANE_DNUMS = lax.GatherDimensionNumbers(
    offset_dims=(), collapsed_slice_dims=(0,), start_index_map=(0,))


def _lane_bcast(v16, el):
    idx = jnp.full((16, 1), el, jnp.int32)
    return lax.gather(v16, idx, _LANE_DNUMS, (1,),
                      mode=lax.GatherScatterMode.PROMISE_IN_BOUNDS)

H1 = 1024
HB = 128
NHB = H1 // HB


def _embed_body(xs_ref, xt_ref, pm_ref, pe_ref, os_ref, ot_ref):
    em = pm_ref[0]
    bm = pm_ref[1]
    ee = pe_ref[0]
    be = pe_ref[1]
    os_ref[...] = jnp.maximum(em * xs_ref[...] + bm, 0.0).T
    ot_ref[...] = jnp.maximum(ee * xt_ref[...] + be, 0.0).T


def _embed(xs2, xt2, pm, pe):
    return pl.pallas_call(
        _embed_body,
        in_specs=[
            pl.BlockSpec((B, G), lambda: (0, 0)),
            pl.BlockSpec((B, G), lambda: (0, 0)),
            pl.BlockSpec(memory_space=pltpu.SMEM),
            pl.BlockSpec(memory_space=pltpu.SMEM),
        ],
        out_specs=[
            pl.BlockSpec((G, B), lambda: (0, 0)),
            pl.BlockSpec((G, B), lambda: (0, 0)),
        ],
        out_shape=[
            jax.ShapeDtypeStruct((G, B), jnp.float32),
            jax.ShapeDtypeStruct((G, B), jnp.float32),
        ],
    )(xs2, xt2, pm, pe)


def _spmm_body(xs, xt, rs, cs, vs, rt, ct, vt, zz, ss, st,
               acc, cv, rv, vv, buf, sem_g, sem_sc, sem_i):
    cid = lax.axis_index("c")
    sid = lax.axis_index("s")

    def run(rows_hbm, cols_hbm, vals_hbm, x_hbm, out_hbm):
        rstart = pl.multiple_of(sid * ROWS0, 8)

        @pl.when(sid < NT - 1)
        def _():
            pltpu.sync_copy(zz.at[pl.ds(rstart, ROWS0)],
                            acc.at[pl.ds(rstart, ROWS0)])

        @pl.when(sid == NT - 1)
        def _():
            base = (NT - 1) * ROWS0
            pltpu.sync_copy(zz.at[pl.ds(base, ROWS_LAST)],
                            acc.at[pl.ds(base, ROWS_LAST)])

        plsc.subcore_barrier()

        def scale_chunk(j, b):
            jv = jnp.full((16,), j, jnp.int32)

            @plsc.parallel_loop(0, CH, 1, unroll=5)
            def _body(e):
                vsp = plsc.load_gather(vv, [jv, jnp.full((16,), e, jnp.int32)])
                for k in range(4):
                    sl = pl.ds(k * 16, 16)
                    buf[b, e, sl] = buf[b, e, sl] * vsp

        def wait_gather(b):
            pltpu.make_async_copy(x_hbm.at[pl.ds(0, CH)], buf.at[b],
                                  sem_g).wait()

        def wait_scatter(b):
            pltpu.make_async_copy(buf.at[b], acc.at[rv.at[0]],
                                  sem_sc).wait()

        def block(blk, carry):
            ci = pltpu.async_copy(cols_hbm.at[sid, blk], cv, sem_i)
            ri = pltpu.async_copy(rows_hbm.at[sid, blk], rv, sem_i)
            vi = pltpu.async_copy(vals_hbm.at[sid, blk], vv, sem_i)
            ci.wait()
            ri.wait()
            vi.wait()
            for p in range(GAHEAD):
                pltpu.async_copy(x_hbm.at[cv.at[p]], buf.at[p], sem_g)

            def it(i, c2):
                for b in range(NBUF):
                    j = NBUF * i + b
                    wait_gather(b)
                    scale_chunk(j, b)
                    pltpu.async_copy(buf.at[b], acc.at[rv.at[j]], sem_sc,
                                     add=True)
                    b2 = (b + GAHEAD) % NBUF
                    if b < NBUF - GAHEAD:
                        @pl.when(i > 0)
                        def _():
                            wait_scatter(b2)
                        pltpu.async_copy(x_hbm.at[cv.at[j + GAHEAD]],
                                         buf.at[b2], sem_g)
                    else:
                        wait_scatter(b2)

                        @pl.when(i < IBC // NBUF - 1)
                        def _():
                            pltpu.async_copy(x_hbm.at[cv.at[j + GAHEAD]],
                                             buf.at[b2], sem_g)
                return c2

            lax.fori_loop(0, IBC // NBUF, it, 0)
            wait_scatter(NBUF - 2)
            wait_scatter(NBUF - 1)
            return carry

        lax.fori_loop(0, NBLK, block, 0)
        plsc.subcore_barrier()

        @pl.when(sid < NT - 1)
        def _():
            pltpu.sync_copy(acc.at[pl.ds(rstart, ROWS0)],
                            out_hbm.at[pl.ds(rstart, ROWS0)])

        @pl.when(sid == NT - 1)
        def _():
            base = (NT - 1) * ROWS0
            pltpu.sync_copy(acc.at[pl.ds(base, ROWS_LAST)],
                            out_hbm.at[pl.ds(base, ROWS_LAST)])

    @pl.when(cid == 0)
    def _():
        run(rs, cs, vs, xs, ss)

    @pl.when(cid == 1)
    def _():
        run(rt, ct, vt, xt, st)


def _spmm(xs0, xt0, rows_s, cols_s, vals_s, rows_t, cols_t, vals_t):
    mesh = plsc.VectorSubcoreMesh(core_axis_name="c", subcore_axis_name="s",
                                  num_cores=2, num_subcores=NT)
    f = pl.kernel(
        _spmm_body,
        out_type=[
            jax.ShapeDtypeStruct((G, B), jnp.float32),
            jax.ShapeDtypeStruct((G, B), jnp.float32),
        ],
        mesh=mesh,
        scratch_types=[
            pltpu.VMEM_SHARED((G, B), jnp.float32),
            pltpu.VMEM((IBC, CH), jnp.int32),
            pltpu.VMEM((IBC, CH), jnp.int32),
            pltpu.VMEM((IBC, CH), jnp.float32),
            pltpu.VMEM((NBUF, CH, B), jnp.float32),
            pltpu.SemaphoreType.DMA,
            pltpu.SemaphoreType.DMA,
            pltpu.SemaphoreType.DMA,
        ],
        compiler_params=pltpu.CompilerParams(needs_layout_passes=False,
                                             use_tc_tiling_on_sc=False),
    )
    zz = jnp.zeros((G, B), jnp.float32)
    return f(xs0, xt0, rows_s, cols_s, vals_s, rows_t, cols_t, vals_t, zz)


def _mlp_body(ss_ref, st_ref, xs0_ref, xt0_ref, w1_ref, b1_ref,
              w2_ref, b2_ref, w3_ref, b3_ref, out_ref, had_scr, h1_scr):
    k = pl.program_id(0)

    @pl.when(k == 0)
    def _():
        beta = 1.0 - ALPHA
        had_scr[...] = ((beta * ss_ref[...] + ALPHA * xs0_ref[...]) *
                        (beta * st_ref[...] + ALPHA * xt0_ref[...]))

    h1_scr[pl.ds(k * HB, HB), :] = jnp.dot(
        w1_ref[...], had_scr[...], preferred_element_type=jnp.float32)

    @pl.when(k == NHB - 1)
    def _():
        h1 = jnp.maximum(h1_scr[...] + b1_ref[...], 0.0)
        h2 = jnp.maximum(
            jnp.dot(w2_ref[...], h1, preferred_element_type=jnp.float32)
            + b2_ref[...], 0.0)
        out_ref[...] = (
            jnp.dot(w3_ref[...], h2, preferred_element_type=jnp.float32)
            + b3_ref[...])


def _mlp(ss, st, xs0, xt0, W1, b1c, W2, b2c, W3, b3c):
    nc = W3.shape[0]
    return pl.pallas_call(
        _mlp_body,
        grid=(NHB,),
        in_specs=[
            pl.BlockSpec((G, B), lambda i: (0, 0)),
            pl.BlockSpec((G, B), lambda i: (0, 0)),
            pl.BlockSpec((G, B), lambda i: (0, 0)),
            pl.BlockSpec((G, B), lambda i: (0, 0)),
            pl.BlockSpec((HB, G), lambda i: (i, 0)),
            pl.BlockSpec((H1, 1), lambda i: (0, 0)),
            pl.BlockSpec((128, H1), lambda i: (0, 0)),
            pl.BlockSpec((128, 1), lambda i: (0, 0)),
            pl.BlockSpec((nc, 128), lambda i: (0, 0)),
            pl.BlockSpec((nc, 1), lambda i: (0, 0)),
        ],
        out_specs=pl.BlockSpec((nc, B), lambda i: (0, 0)),
        out_shape=jax.ShapeDtypeStruct((nc, B), jnp.float32),
        scratch_shapes=[pltpu.VMEM((G, B), jnp.float32),
                        pltpu.VMEM((H1, B), jnp.float32)],
    )(ss, st, xs0, xt0, W1, b1c, W2, b2c, W3, b3c)


def kernel(x_sample, x_TF, adj_idx, adj_val, adj_t_idx, adj_t_val,
           emb_mut, bias_mut, emb_exp, bias_exp,
           W1, b1, W2, b2, W3, b3):
    xs2 = x_sample.reshape(B, G)
    xt2 = x_TF.reshape(B, G)
    pm = jnp.concatenate([emb_mut, bias_mut]).astype(jnp.float32)
    pe = jnp.concatenate([emb_exp, bias_exp]).astype(jnp.float32)
    xs0, xt0 = _embed(xs2, xt2, pm, pe)

    esh = (NT, NBLK, IBC, CH)
    rows_s = adj_idx[0].astype(jnp.int32).reshape(esh)
    cols_s = adj_idx[1].astype(jnp.int32).reshape(esh)
    rows_t = adj_t_idx[0].astype(jnp.int32).reshape(esh)
    cols_t = adj_t_idx[1].astype(jnp.int32).reshape(esh)
    ss, st = _spmm(xs0, xt0, rows_s, cols_s, adj_val.reshape(esh),
                   rows_t, cols_t, adj_t_val.reshape(esh))

    out = _mlp(ss, st, xs0, xt0,
               W1, b1.reshape(-1, 1), W2, b2.reshape(-1, 1),
               W3, b3.reshape(-1, 1))
    return out.T


# speedup vs baseline: 1.4414x; 1.1977x over previous
import functools

import jax
import jax.numpy as jnp
from jax import lax
from jax.experimental import pallas as pl
from jax.experimental.pallas import tpu as pltpu
from jax.experimental.pallas import tpu_sc as plsc

G = 10000
B = 64
NNZ = 640000
ALPHA = 0.1

NT = 16
EPT = NNZ // NT
CH = 80
NBUF = 5
GAHEAD = 3
IBC = 100
NBLK = EPT // (IBC * CH)
ROWS0 = 632
ROWS_LAST = G - (NT - 1) * ROWS0
ZR = 320

_LANE_DNUMS = lax.GatherDimensionNumbers(
    offset_dims=(), collapsed_slice_dims=(0,), start_index_map=(0,))


def _lane_bcast(v16, el):
    idx = jnp.full((16, 1), el, jnp.int32)
    return lax.gather(v16, idx, _LANE_DNUMS, (1,),
                      mode=lax.GatherScatterMode.PROMISE_IN_BOUNDS)

H1 = 1024
HB = 128
NHB = H1 // HB


def _embed_body(xs_ref, xt_ref, pm_ref, pe_ref, os_ref, ot_ref):
    em = pm_ref[0]
    bm = pm_ref[1]
    ee = pe_ref[0]
    be = pe_ref[1]
    os_ref[...] = jnp.maximum(em * xs_ref[...] + bm, 0.0).T
    ot_ref[...] = jnp.maximum(ee * xt_ref[...] + be, 0.0).T


def _embed(xs2, xt2, pm, pe):
    return pl.pallas_call(
        _embed_body,
        in_specs=[
            pl.BlockSpec((B, G), lambda: (0, 0)),
            pl.BlockSpec((B, G), lambda: (0, 0)),
            pl.BlockSpec(memory_space=pltpu.SMEM),
            pl.BlockSpec(memory_space=pltpu.SMEM),
        ],
        out_specs=[
            pl.BlockSpec((G, B), lambda: (0, 0)),
            pl.BlockSpec((G, B), lambda: (0, 0)),
        ],
        out_shape=[
            jax.ShapeDtypeStruct((G, B), jnp.float32),
            jax.ShapeDtypeStruct((G, B), jnp.float32),
        ],
    )(xs2, xt2, pm, pe)


def _spmm_body(xs, xt, rs, cs, vs, rt, ct, vt, zz, ss, st,
               acc, cv, rv, vv, buf, sem_g, sem_sc, sem_i):
    cid = lax.axis_index("c")
    sid = lax.axis_index("s")

    def run(rows_hbm, cols_hbm, vals_hbm, x_hbm, out_hbm):
        rstart = pl.multiple_of(sid * ROWS0, 8)

        @pl.when(sid < NT - 1)
        def _():
            pltpu.sync_copy(zz.at[pl.ds(rstart, ROWS0)],
                            acc.at[pl.ds(rstart, ROWS0)])

        @pl.when(sid == NT - 1)
        def _():
            base = (NT - 1) * ROWS0
            pltpu.sync_copy(zz.at[pl.ds(base, ROWS_LAST)],
                            acc.at[pl.ds(base, ROWS_LAST)])

        plsc.subcore_barrier()

        def scale_chunk(j, b):
            jv = jnp.full((16,), j, jnp.int32)

            @plsc.parallel_loop(0, CH, 1, unroll=5)
            def _body(e):
                vsp = plsc.load_gather(vv, [jv, jnp.full((16,), e, jnp.int32)])
                for k in range(4):
                    sl = pl.ds(k * 16, 16)
                    buf[b, e, sl] = buf[b, e, sl] * vsp

        def wait_gather(b):
            pltpu.make_async_copy(x_hbm.at[pl.ds(0, CH)], buf.at[b],
                                  sem_g).wait()

        def wait_scatter(b):
            pltpu.make_async_copy(buf.at[b], acc.at[rv.at[0]],
                                  sem_sc).wait()

        def block(blk, carry):
            ci = pltpu.async_copy(cols_hbm.at[sid, blk], cv, sem_i)
            ri = pltpu.async_copy(rows_hbm.at[sid, blk], rv, sem_i)
            vi = pltpu.async_copy(vals_hbm.at[sid, blk], vv, sem_i)
            ci.wait()
            ri.wait()
            vi.wait()
            for p in range(GAHEAD):
                pltpu.async_copy(x_hbm.at[cv.at[p]], buf.at[p], sem_g)

            def it(i, c2):
                for b in range(NBUF):
                    j = NBUF * i + b
                    wait_gather(b)
                    scale_chunk(j, b)
                    pltpu.async_copy(buf.at[b], acc.at[rv.at[j]], sem_sc,
                                     add=True)
                    b2 = (b + GAHEAD) % NBUF
                    if b < NBUF - GAHEAD:
                        @pl.when(i > 0)
                        def _():
                            wait_scatter(b2)
                        pltpu.async_copy(x_hbm.at[cv.at[j + GAHEAD]],
                                         buf.at[b2], sem_g)
                    else:
                        wait_scatter(b2)

                        @pl.when(i < IBC // NBUF - 1)
                        def _():
                            pltpu.async_copy(x_hbm.at[cv.at[j + GAHEAD]],
                                             buf.at[b2], sem_g)
                return c2

            lax.fori_loop(0, IBC // NBUF, it, 0)
            wait_scatter(NBUF - 2)
            wait_scatter(NBUF - 1)
            return carry

        lax.fori_loop(0, NBLK, block, 0)
        plsc.subcore_barrier()

        @pl.when(sid < NT - 1)
        def _():
            pltpu.sync_copy(acc.at[pl.ds(rstart, ROWS0)],
                            out_hbm.at[pl.ds(rstart, ROWS0)])

        @pl.when(sid == NT - 1)
        def _():
            base = (NT - 1) * ROWS0
            pltpu.sync_copy(acc.at[pl.ds(base, ROWS_LAST)],
                            out_hbm.at[pl.ds(base, ROWS_LAST)])

    @pl.when(cid == 0)
    def _():
        run(rs, cs, vs, xs, ss)

    @pl.when(cid == 1)
    def _():
        run(rt, ct, vt, xt, st)


def _spmm(xs0, xt0, rows_s, cols_s, vals_s, rows_t, cols_t, vals_t):
    mesh = plsc.VectorSubcoreMesh(core_axis_name="c", subcore_axis_name="s",
                                  num_cores=2, num_subcores=NT)
    f = pl.kernel(
        _spmm_body,
        out_type=[
            jax.ShapeDtypeStruct((G, B), jnp.float32),
            jax.ShapeDtypeStruct((G, B), jnp.float32),
        ],
        mesh=mesh,
        scratch_types=[
            pltpu.VMEM_SHARED((G, B), jnp.float32),
            pltpu.VMEM((IBC, CH), jnp.int32),
            pltpu.VMEM((IBC, CH), jnp.int32),
            pltpu.VMEM((IBC, CH), jnp.float32),
            pltpu.VMEM((NBUF, CH, B), jnp.float32),
            pltpu.SemaphoreType.DMA,
            pltpu.SemaphoreType.DMA,
            pltpu.SemaphoreType.DMA,
        ],
        compiler_params=pltpu.CompilerParams(needs_layout_passes=False,
                                             use_tc_tiling_on_sc=False),
    )
    zz = jnp.zeros((G, B), jnp.float32)
    return f(xs0, xt0, rows_s, cols_s, vals_s, rows_t, cols_t, vals_t, zz)


def _mlp_body(ss_ref, st_ref, xs0_ref, xt0_ref, w1_ref, b1_ref,
              w2_ref, b2_ref, w3_ref, b3_ref, out_ref, had_scr, h1_scr):
    k = pl.program_id(0)

    @pl.when(k == 0)
    def _():
        beta = 1.0 - ALPHA
        had_scr[...] = ((beta * ss_ref[...] + ALPHA * xs0_ref[...]) *
                        (beta * st_ref[...] + ALPHA * xt0_ref[...]))

    h1_scr[pl.ds(k * HB, HB), :] = jnp.dot(
        w1_ref[...], had_scr[...], preferred_element_type=jnp.float32)

    @pl.when(k == NHB - 1)
    def _():
        h1 = jnp.maximum(h1_scr[...] + b1_ref[...], 0.0)
        h2 = jnp.maximum(
            jnp.dot(w2_ref[...], h1, preferred_element_type=jnp.float32)
            + b2_ref[...], 0.0)
        out_ref[...] = (
            jnp.dot(w3_ref[...], h2, preferred_element_type=jnp.float32)
            + b3_ref[...])


def _mlp(ss, st, xs0, xt0, W1, b1c, W2, b2c, W3, b3c):
    nc = W3.shape[0]
    return pl.pallas_call(
        _mlp_body,
        grid=(NHB,),
        in_specs=[
            pl.BlockSpec((G, B), lambda i: (0, 0)),
            pl.BlockSpec((G, B), lambda i: (0, 0)),
            pl.BlockSpec((G, B), lambda i: (0, 0)),
            pl.BlockSpec((G, B), lambda i: (0, 0)),
            pl.BlockSpec((HB, G), lambda i: (i, 0)),
            pl.BlockSpec((H1, 1), lambda i: (0, 0)),
            pl.BlockSpec((128, H1), lambda i: (0, 0)),
            pl.BlockSpec((128, 1), lambda i: (0, 0)),
            pl.BlockSpec((nc, 128), lambda i: (0, 0)),
            pl.BlockSpec((nc, 1), lambda i: (0, 0)),
        ],
        out_specs=pl.BlockSpec((nc, B), lambda i: (0, 0)),
        out_shape=jax.ShapeDtypeStruct((nc, B), jnp.float32),
        scratch_shapes=[pltpu.VMEM((G, B), jnp.float32),
                        pltpu.VMEM((H1, B), jnp.float32)],
    )(ss, st, xs0, xt0, W1, b1c, W2, b2c, W3, b3c)


def kernel(x_sample, x_TF, adj_idx, adj_val, adj_t_idx, adj_t_val,
           emb_mut, bias_mut, emb_exp, bias_exp,
           W1, b1, W2, b2, W3, b3):
    xs2 = x_sample.reshape(B, G)
    xt2 = x_TF.reshape(B, G)
    pm = jnp.concatenate([emb_mut, bias_mut]).astype(jnp.float32)
    pe = jnp.concatenate([emb_exp, bias_exp]).astype(jnp.float32)
    xs0, xt0 = _embed(xs2, xt2, pm, pe)

    esh = (NT, NBLK, IBC, CH)
    rows_s = adj_idx[0].astype(jnp.int32).reshape(esh)
    cols_s = adj_idx[1].astype(jnp.int32).reshape(esh)
    rows_t = adj_t_idx[0].astype(jnp.int32).reshape(esh)
    cols_t = adj_t_idx[1].astype(jnp.int32).reshape(esh)
    ss, st = _spmm(xs0, xt0, rows_s, cols_s, adj_val.reshape(esh),
                   rows_t, cols_t, adj_t_val.reshape(esh))

    out = _mlp(ss, st, xs0, xt0,
               W1, b1.reshape(-1, 1), W2, b2.reshape(-1, 1),
               W3, b3.reshape(-1, 1))
    return out.T


# speedup vs baseline: 1.4690x; 1.0192x over previous
import functools

import jax
import jax.numpy as jnp
from jax import lax
from jax.experimental import pallas as pl
from jax.experimental.pallas import tpu as pltpu
from jax.experimental.pallas import tpu_sc as plsc

G = 10000
B = 64
NNZ = 640000
ALPHA = 0.1

NT = 16
EPT = NNZ // NT
CH = 80
NBUF = 10
GAHEAD = 8
IBC = 100
NBLK = EPT // (IBC * CH)
ROWS0 = 632
ROWS_LAST = G - (NT - 1) * ROWS0
ZR = 320

_LANE_DNUMS = lax.GatherDimensionNumbers(
    offset_dims=(), collapsed_slice_dims=(0,), start_index_map=(0,))


def _lane_bcast(v16, el):
    idx = jnp.full((16, 1), el, jnp.int32)
    return lax.gather(v16, idx, _LANE_DNUMS, (1,),
                      mode=lax.GatherScatterMode.PROMISE_IN_BOUNDS)

H1 = 1024
HB = 128
NHB = H1 // HB


def _embed_body(xs_ref, xt_ref, pm_ref, pe_ref, os_ref, ot_ref):
    em = pm_ref[0]
    bm = pm_ref[1]
    ee = pe_ref[0]
    be = pe_ref[1]
    os_ref[...] = jnp.maximum(em * xs_ref[...] + bm, 0.0).T
    ot_ref[...] = jnp.maximum(ee * xt_ref[...] + be, 0.0).T


def _embed(xs2, xt2, pm, pe):
    return pl.pallas_call(
        _embed_body,
        in_specs=[
            pl.BlockSpec((B, G), lambda: (0, 0)),
            pl.BlockSpec((B, G), lambda: (0, 0)),
            pl.BlockSpec(memory_space=pltpu.SMEM),
            pl.BlockSpec(memory_space=pltpu.SMEM),
        ],
        out_specs=[
            pl.BlockSpec((G, B), lambda: (0, 0)),
            pl.BlockSpec((G, B), lambda: (0, 0)),
        ],
        out_shape=[
            jax.ShapeDtypeStruct((G, B), jnp.float32),
            jax.ShapeDtypeStruct((G, B), jnp.float32),
        ],
    )(xs2, xt2, pm, pe)


def _spmm_body(xs, xt, rs, cs, vs, rt, ct, vt, zz, ss, st,
               acc, cv, rv, vv, buf, sem_g, sem_sc, sem_i):
    cid = lax.axis_index("c")
    sid = lax.axis_index("s")

    def run(rows_hbm, cols_hbm, vals_hbm, x_hbm, out_hbm):
        rstart = pl.multiple_of(sid * ROWS0, 8)

        @pl.when(sid < NT - 1)
        def _():
            pltpu.sync_copy(zz.at[pl.ds(rstart, ROWS0)],
                            acc.at[pl.ds(rstart, ROWS0)])

        @pl.when(sid == NT - 1)
        def _():
            base = (NT - 1) * ROWS0
            pltpu.sync_copy(zz.at[pl.ds(base, ROWS_LAST)],
                            acc.at[pl.ds(base, ROWS_LAST)])

        plsc.subcore_barrier()

        def scale_chunk(j, b):
            jv = jnp.full((16,), j, jnp.int32)

            @plsc.parallel_loop(0, CH, 1, unroll=5)
            def _body(e):
                vsp = plsc.load_gather(vv, [jv, jnp.full((16,), e, jnp.int32)])
                for k in range(4):
                    sl = pl.ds(k * 16, 16)
                    buf[b, e, sl] = buf[b, e, sl] * vsp

        def wait_gather(b):
            pltpu.make_async_copy(x_hbm.at[pl.ds(0, CH)], buf.at[b],
                                  sem_g).wait()

        def wait_scatter(b):
            pltpu.make_async_copy(buf.at[b], acc.at[rv.at[0]],
                                  sem_sc).wait()

        def block(blk, carry):
            ci = pltpu.async_copy(cols_hbm.at[sid, blk], cv, sem_i)
            ri = pltpu.async_copy(rows_hbm.at[sid, blk], rv, sem_i)
            vi = pltpu.async_copy(vals_hbm.at[sid, blk], vv, sem_i)
            ci.wait()
            ri.wait()
            vi.wait()
            for p in range(GAHEAD):
                pltpu.async_copy(x_hbm.at[cv.at[p]], buf.at[p], sem_g)

            def it(i, c2):
                for b in range(NBUF):
                    j = NBUF * i + b
                    wait_gather(b)
                    scale_chunk(j, b)
                    pltpu.async_copy(buf.at[b], acc.at[rv.at[j]], sem_sc,
                                     add=True)
                    b2 = (b + GAHEAD) % NBUF
                    if b < NBUF - GAHEAD:
                        @pl.when(i > 0)
                        def _():
                            wait_scatter(b2)
                        pltpu.async_copy(x_hbm.at[cv.at[j + GAHEAD]],
                                         buf.at[b2], sem_g)
                    else:
                        wait_scatter(b2)

                        @pl.when(i < IBC // NBUF - 1)
                        def _():
                            pltpu.async_copy(x_hbm.at[cv.at[j + GAHEAD]],
                                             buf.at[b2], sem_g)
                return c2

            lax.fori_loop(0, IBC // NBUF, it, 0)
            wait_scatter(NBUF - 2)
            wait_scatter(NBUF - 1)
            return carry

        lax.fori_loop(0, NBLK, block, 0)
        plsc.subcore_barrier()

        @pl.when(sid < NT - 1)
        def _():
            pltpu.sync_copy(acc.at[pl.ds(rstart, ROWS0)],
                            out_hbm.at[pl.ds(rstart, ROWS0)])

        @pl.when(sid == NT - 1)
        def _():
            base = (NT - 1) * ROWS0
            pltpu.sync_copy(acc.at[pl.ds(base, ROWS_LAST)],
                            out_hbm.at[pl.ds(base, ROWS_LAST)])

    @pl.when(cid == 0)
    def _():
        run(rs, cs, vs, xs, ss)

    @pl.when(cid == 1)
    def _():
        run(rt, ct, vt, xt, st)


def _spmm(xs0, xt0, rows_s, cols_s, vals_s, rows_t, cols_t, vals_t):
    mesh = plsc.VectorSubcoreMesh(core_axis_name="c", subcore_axis_name="s",
                                  num_cores=2, num_subcores=NT)
    f = pl.kernel(
        _spmm_body,
        out_type=[
            jax.ShapeDtypeStruct((G, B), jnp.float32),
            jax.ShapeDtypeStruct((G, B), jnp.float32),
        ],
        mesh=mesh,
        scratch_types=[
            pltpu.VMEM_SHARED((G, B), jnp.float32),
            pltpu.VMEM((IBC, CH), jnp.int32),
            pltpu.VMEM((IBC, CH), jnp.int32),
            pltpu.VMEM((IBC, CH), jnp.float32),
            pltpu.VMEM((NBUF, CH, B), jnp.float32),
            pltpu.SemaphoreType.DMA,
            pltpu.SemaphoreType.DMA,
            pltpu.SemaphoreType.DMA,
        ],
        compiler_params=pltpu.CompilerParams(needs_layout_passes=False,
                                             use_tc_tiling_on_sc=False),
    )
    zz = jnp.zeros((G, B), jnp.float32)
    return f(xs0, xt0, rows_s, cols_s, vals_s, rows_t, cols_t, vals_t, zz)


def _mlp_body(ss_ref, st_ref, xs0_ref, xt0_ref, w1_ref, b1_ref,
              w2_ref, b2_ref, w3_ref, b3_ref, out_ref, had_scr, h1_scr):
    k = pl.program_id(0)

    @pl.when(k == 0)
    def _():
        beta = 1.0 - ALPHA
        had_scr[...] = ((beta * ss_ref[...] + ALPHA * xs0_ref[...]) *
                        (beta * st_ref[...] + ALPHA * xt0_ref[...]))

    h1_scr[pl.ds(k * HB, HB), :] = jnp.dot(
        w1_ref[...], had_scr[...], preferred_element_type=jnp.float32)

    @pl.when(k == NHB - 1)
    def _():
        h1 = jnp.maximum(h1_scr[...] + b1_ref[...], 0.0)
        h2 = jnp.maximum(
            jnp.dot(w2_ref[...], h1, preferred_element_type=jnp.float32)
            + b2_ref[...], 0.0)
        out_ref[...] = (
            jnp.dot(w3_ref[...], h2, preferred_element_type=jnp.float32)
            + b3_ref[...])


def _mlp(ss, st, xs0, xt0, W1, b1c, W2, b2c, W3, b3c):
    nc = W3.shape[0]
    return pl.pallas_call(
        _mlp_body,
        grid=(NHB,),
        in_specs=[
            pl.BlockSpec((G, B), lambda i: (0, 0)),
            pl.BlockSpec((G, B), lambda i: (0, 0)),
            pl.BlockSpec((G, B), lambda i: (0, 0)),
            pl.BlockSpec((G, B), lambda i: (0, 0)),
            pl.BlockSpec((HB, G), lambda i: (i, 0)),
            pl.BlockSpec((H1, 1), lambda i: (0, 0)),
            pl.BlockSpec((128, H1), lambda i: (0, 0)),
            pl.BlockSpec((128, 1), lambda i: (0, 0)),
            pl.BlockSpec((nc, 128), lambda i: (0, 0)),
            pl.BlockSpec((nc, 1), lambda i: (0, 0)),
        ],
        out_specs=pl.BlockSpec((nc, B), lambda i: (0, 0)),
        out_shape=jax.ShapeDtypeStruct((nc, B), jnp.float32),
        scratch_shapes=[pltpu.VMEM((G, B), jnp.float32),
                        pltpu.VMEM((H1, B), jnp.float32)],
    )(ss, st, xs0, xt0, W1, b1c, W2, b2c, W3, b3c)


def kernel(x_sample, x_TF, adj_idx, adj_val, adj_t_idx, adj_t_val,
           emb_mut, bias_mut, emb_exp, bias_exp,
           W1, b1, W2, b2, W3, b3):
    xs2 = x_sample.reshape(B, G)
    xt2 = x_TF.reshape(B, G)
    pm = jnp.concatenate([emb_mut, bias_mut]).astype(jnp.float32)
    pe = jnp.concatenate([emb_exp, bias_exp]).astype(jnp.float32)
    xs0, xt0 = _embed(xs2, xt2, pm, pe)

    esh = (NT, NBLK, IBC, CH)
    rows_s = adj_idx[0].astype(jnp.int32).reshape(esh)
    cols_s = adj_idx[1].astype(jnp.int32).reshape(esh)
    rows_t = adj_t_idx[0].astype(jnp.int32).reshape(esh)
    cols_t = adj_t_idx[1].astype(jnp.int32).reshape(esh)
    ss, st = _spmm(xs0, xt0, rows_s, cols_s, adj_val.reshape(esh),
                   rows_t, cols_t, adj_t_val.reshape(esh))

    out = _mlp(ss, st, xs0, xt0,
               W1, b1.reshape(-1, 1), W2, b2.reshape(-1, 1),
               W3, b3.reshape(-1, 1))
    return out.T


# speedup vs baseline: 1.5038x; 1.0237x over previous
import functools

import jax
import jax.numpy as jnp
from jax import lax
from jax.experimental import pallas as pl
from jax.experimental.pallas import tpu as pltpu
from jax.experimental.pallas import tpu_sc as plsc

G = 10000
B = 64
NNZ = 640000
ALPHA = 0.1

NT = 16
EPT = NNZ // NT
CH = 80
NBUF = 10
GAHEAD = 8
IBC = 100
NBLK = EPT // (IBC * CH)
ROWS0 = 632
ROWS_LAST = G - (NT - 1) * ROWS0
ZR = 320

_LANE_DNUMS = lax.GatherDimensionNumbers(
    offset_dims=(), collapsed_slice_dims=(0,), start_index_map=(0,))


def _lane_bcast(v16, el):
    idx = jnp.full((16, 1), el, jnp.int32)
    return lax.gather(v16, idx, _LANE_DNUMS, (1,),
                      mode=lax.GatherScatterMode.PROMISE_IN_BOUNDS)

H1 = 1024
HB = 128
NHB = H1 // HB


def _embed_body(xs_ref, xt_ref, pm_ref, pe_ref, os_ref, ot_ref):
    em = pm_ref[0]
    bm = pm_ref[1]
    ee = pe_ref[0]
    be = pe_ref[1]
    os_ref[...] = jnp.maximum(em * xs_ref[...] + bm, 0.0).T
    ot_ref[...] = jnp.maximum(ee * xt_ref[...] + be, 0.0).T


def _embed(xs2, xt2, pm, pe):
    return pl.pallas_call(
        _embed_body,
        in_specs=[
            pl.BlockSpec((B, G), lambda: (0, 0)),
            pl.BlockSpec((B, G), lambda: (0, 0)),
            pl.BlockSpec(memory_space=pltpu.SMEM),
            pl.BlockSpec(memory_space=pltpu.SMEM),
        ],
        out_specs=[
            pl.BlockSpec((G, B), lambda: (0, 0)),
            pl.BlockSpec((G, B), lambda: (0, 0)),
        ],
        out_shape=[
            jax.ShapeDtypeStruct((G, B), jnp.float32),
            jax.ShapeDtypeStruct((G, B), jnp.float32),
        ],
    )(xs2, xt2, pm, pe)


def _spmm_body(xs, xt, rs, cs, vs, rt, ct, vt, zz, ss, st,
               acc, cv, rv, vv, buf, sem_g, sem_sc, sem_i):
    cid = lax.axis_index("c")
    sid = lax.axis_index("s")

    def run(rows_hbm, cols_hbm, vals_hbm, x_hbm, out_hbm):
        rstart = pl.multiple_of(sid * ROWS0, 8)

        @pl.when(sid < NT - 1)
        def _():
            pltpu.sync_copy(zz.at[pl.ds(rstart, ROWS0)],
                            acc.at[pl.ds(rstart, ROWS0)])

        @pl.when(sid == NT - 1)
        def _():
            base = (NT - 1) * ROWS0
            pltpu.sync_copy(zz.at[pl.ds(base, ROWS_LAST)],
                            acc.at[pl.ds(base, ROWS_LAST)])

        plsc.subcore_barrier()

        def scale_chunk(j, b):
            jv = jnp.full((16,), j, jnp.int32)

            @plsc.parallel_loop(0, CH, 1, unroll=5)
            def _body(e):
                vsp = plsc.load_gather(vv, [jv, jnp.full((16,), e, jnp.int32)])
                for k in range(4):
                    sl = pl.ds(k * 16, 16)
                    buf[b, e, sl] = buf[b, e, sl] * vsp

        def wait_gather(b):
            pltpu.make_async_copy(x_hbm.at[pl.ds(0, CH)], buf.at[b],
                                  sem_g).wait()

        def wait_scatter(b):
            pltpu.make_async_copy(buf.at[b], acc.at[rv.at[0]],
                                  sem_sc).wait()

        def block(blk, carry):
            ci = pltpu.async_copy(cols_hbm.at[sid, blk], cv, sem_i)
            ri = pltpu.async_copy(rows_hbm.at[sid, blk], rv, sem_i)
            vi = pltpu.async_copy(vals_hbm.at[sid, blk], vv, sem_i)
            ci.wait()
            ri.wait()
            vi.wait()
            for p in range(GAHEAD):
                pltpu.async_copy(x_hbm.at[cv.at[p]], buf.at[p], sem_g)

            def it(i, c2):
                for b in range(NBUF):
                    j = NBUF * i + b
                    wait_gather(b)
                    scale_chunk(j, b)
                    pltpu.async_copy(buf.at[b], acc.at[rv.at[j]], sem_sc,
                                     add=True)
                    b2 = (b + GAHEAD) % NBUF
                    if b < NBUF - GAHEAD:
                        @pl.when(i > 0)
                        def _():
                            wait_scatter(b2)
                        pltpu.async_copy(x_hbm.at[cv.at[j + GAHEAD]],
                                         buf.at[b2], sem_g)
                    else:
                        wait_scatter(b2)

                        @pl.when(i < IBC // NBUF - 1)
                        def _():
                            pltpu.async_copy(x_hbm.at[cv.at[j + GAHEAD]],
                                             buf.at[b2], sem_g)
                return c2

            lax.fori_loop(0, IBC // NBUF, it, 0)
            wait_scatter(NBUF - 2)
            wait_scatter(NBUF - 1)
            return carry

        lax.fori_loop(0, NBLK, block, 0)
        plsc.subcore_barrier()

        @pl.when(sid < NT - 1)
        def _():
            pltpu.sync_copy(acc.at[pl.ds(rstart, ROWS0)],
                            out_hbm.at[pl.ds(rstart, ROWS0)])

        @pl.when(sid == NT - 1)
        def _():
            base = (NT - 1) * ROWS0
            pltpu.sync_copy(acc.at[pl.ds(base, ROWS_LAST)],
                            out_hbm.at[pl.ds(base, ROWS_LAST)])

    @pl.when(cid == 0)
    def _():
        run(rs, cs, vs, xs, ss)

    @pl.when(cid == 1)
    def _():
        run(rt, ct, vt, xt, st)


def _spmm(xs0, xt0, rows_s, cols_s, vals_s, rows_t, cols_t, vals_t):
    mesh = plsc.VectorSubcoreMesh(core_axis_name="c", subcore_axis_name="s",
                                  num_cores=2, num_subcores=NT)
    f = pl.kernel(
        _spmm_body,
        out_type=[
            jax.ShapeDtypeStruct((G, B), jnp.float32),
            jax.ShapeDtypeStruct((G, B), jnp.float32),
        ],
        mesh=mesh,
        scratch_types=[
            pltpu.VMEM_SHARED((G, B), jnp.float32),
            pltpu.VMEM((IBC, CH), jnp.int32),
            pltpu.VMEM((IBC, CH), jnp.int32),
            pltpu.VMEM((IBC, CH), jnp.float32),
            pltpu.VMEM((NBUF, CH, B), jnp.float32),
            pltpu.SemaphoreType.DMA,
            pltpu.SemaphoreType.DMA,
            pltpu.SemaphoreType.DMA,
        ],
        compiler_params=pltpu.CompilerParams(needs_layout_passes=False,
                                             use_tc_tiling_on_sc=False),
    )
    zz = jnp.zeros((G, B), jnp.float32)
    return f(xs0, xt0, rows_s, cols_s, vals_s, rows_t, cols_t, vals_t, zz)


def _mlp_body(ss_ref, st_ref, xs0_ref, xt0_ref, w1_ref, b1_ref,
              w2_ref, b2_ref, w3_ref, b3_ref, out_ref, had_scr, h1_scr):
    k = pl.program_id(0)

    @pl.when(k == 0)
    def _():
        beta = 1.0 - ALPHA
        had_scr[...] = ((beta * ss_ref[...] + ALPHA * xs0_ref[...]) *
                        (beta * st_ref[...] + ALPHA * xt0_ref[...]))

    h1_scr[pl.ds(k * HB, HB), :] = lax.dot_general(
        w1_ref[...], had_scr[...], (((0,), (0,)), ((), ())),
        preferred_element_type=jnp.float32)

    @pl.when(k == NHB - 1)
    def _():
        h1 = jnp.maximum(h1_scr[...] + b1_ref[...], 0.0)
        h2 = jnp.maximum(
            jnp.dot(w2_ref[...], h1, preferred_element_type=jnp.float32)
            + b2_ref[...], 0.0)
        out_ref[...] = (
            jnp.dot(w3_ref[...], h2, preferred_element_type=jnp.float32)
            + b3_ref[...])


def _mlp(ss, st, xs0, xt0, W1t, b1c, W2, b2c, W3, b3c):
    nc = W3.shape[0]
    return pl.pallas_call(
        _mlp_body,
        grid=(NHB,),
        in_specs=[
            pl.BlockSpec((G, B), lambda i: (0, 0)),
            pl.BlockSpec((G, B), lambda i: (0, 0)),
            pl.BlockSpec((G, B), lambda i: (0, 0)),
            pl.BlockSpec((G, B), lambda i: (0, 0)),
            pl.BlockSpec((G, HB), lambda i: (0, i)),
            pl.BlockSpec((H1, 1), lambda i: (0, 0)),
            pl.BlockSpec((128, H1), lambda i: (0, 0)),
            pl.BlockSpec((128, 1), lambda i: (0, 0)),
            pl.BlockSpec((nc, 128), lambda i: (0, 0)),
            pl.BlockSpec((nc, 1), lambda i: (0, 0)),
        ],
        out_specs=pl.BlockSpec((nc, B), lambda i: (0, 0)),
        out_shape=jax.ShapeDtypeStruct((nc, B), jnp.float32),
        scratch_shapes=[pltpu.VMEM((G, B), jnp.float32),
                        pltpu.VMEM((H1, B), jnp.float32)],
    )(ss, st, xs0, xt0, W1t, b1c, W2, b2c, W3, b3c)


def kernel(x_sample, x_TF, adj_idx, adj_val, adj_t_idx, adj_t_val,
           emb_mut, bias_mut, emb_exp, bias_exp,
           W1, b1, W2, b2, W3, b3):
    xs2 = x_sample.reshape(B, G)
    xt2 = x_TF.reshape(B, G)
    pm = jnp.concatenate([emb_mut, bias_mut]).astype(jnp.float32)
    pe = jnp.concatenate([emb_exp, bias_exp]).astype(jnp.float32)
    xs0, xt0 = _embed(xs2, xt2, pm, pe)

    esh = (NT, NBLK, IBC, CH)
    rows_s = adj_idx[0].astype(jnp.int32).reshape(esh)
    cols_s = adj_idx[1].astype(jnp.int32).reshape(esh)
    rows_t = adj_t_idx[0].astype(jnp.int32).reshape(esh)
    cols_t = adj_t_idx[1].astype(jnp.int32).reshape(esh)
    ss, st = _spmm(xs0, xt0, rows_s, cols_s, adj_val.reshape(esh),
                   rows_t, cols_t, adj_t_val.reshape(esh))

    out = _mlp(ss, st, xs0, xt0,
               W1.T, b1.reshape(-1, 1), W2, b2.reshape(-1, 1),
               W3, b3.reshape(-1, 1))
    return out.T
